# baseline (speedup 1.0000x reference)
import jax
import jax.numpy as jnp
from jax import lax
from jax.experimental import pallas as pl
from jax.experimental.pallas import tpu as pltpu
from jax.experimental.pallas import tpu_sc as plsc

N_ATOMS = 1000000
R = 8
G = 100000
L = 3
O = 4
ZIDX = 2 * N_ATOMS
T2 = ZIDX + 8

NC = 2
NS = 16
NW = NC * NS
T = (R * G) // NW
K = 12480
NCH = 2
NGRP = K // 16
U = 6
TAIL = T - NCH * K
TGRP = 3
TB = TGRP * 16


def _sc_body(tbl2_hbm, s0_hbm, s1_hbm, s2_hbm, part_hbm,
             s0_v, s1_v, s2_v, p0_v, p1_v, p2_v,
             st_v, pt_v, acc_v, sem):
    wid = lax.axis_index("s") * NC + lax.axis_index("c")
    g0 = wid * T

    iota = lax.iota(jnp.int32, 16)
    zero_v = jnp.zeros((16,), jnp.float32)
    zero_i = jnp.zeros((16,), jnp.int32)

    st_v[pl.ds(L * TB - 16, 16)] = zero_i

    def chunk_body(ci, acc):
        b = g0 + ci * K
        pltpu.sync_copy(s0_hbm.at[pl.ds(b, K)], s0_v)
        pltpu.sync_copy(s1_hbm.at[pl.ds(b, K)], s1_v)
        pltpu.sync_copy(s2_hbm.at[pl.ds(b, K)], s2_v)
        pltpu.async_copy(tbl2_hbm.at[s0_v], p0_v, sem)
        pltpu.async_copy(tbl2_hbm.at[s1_v], p1_v, sem)
        pltpu.async_copy(tbl2_hbm.at[s2_v], p2_v, sem)
        pltpu.make_async_copy(s0_hbm.at[pl.ds(0, K)], p0_v, sem).wait()
        pltpu.make_async_copy(s0_hbm.at[pl.ds(0, K)], p1_v, sem).wait()
        pltpu.make_async_copy(s0_hbm.at[pl.ds(0, K)], p2_v, sem).wait()

        def grp(i, a):
            for u in range(U):
                ds_ = pl.ds((i * U + u) * 16, 16)
                a = a + p0_v[ds_] * p1_v[ds_] * p2_v[ds_]
            return a

        return lax.fori_loop(0, NGRP // U, grp, acc)

    acc = lax.fori_loop(0, NCH, chunk_body, zero_v)

    b = g0 + NCH * K
    pltpu.sync_copy(s0_hbm.at[pl.ds(b, TAIL)], st_v.at[pl.ds(0, TAIL)])
    pltpu.sync_copy(s1_hbm.at[pl.ds(b, TAIL)], st_v.at[pl.ds(TB, TAIL)])
    pltpu.sync_copy(s2_hbm.at[pl.ds(b, TAIL)], st_v.at[pl.ds(2 * TB, TAIL)])
    pltpu.async_copy(tbl2_hbm.at[st_v], pt_v, sem).wait()
    for j in range(TGRP):
        ds_ = pl.ds(j * 16, 16)
        prod = (pt_v[ds_] * pt_v[pl.ds(TB + j * 16, 16)]
                * pt_v[pl.ds(2 * TB + j * 16, 16)])
        valid = (j * 16 + iota) < TAIL
        acc = acc + jnp.where(valid, prod, zero_v)

    acc_v[...] = acc
    pltpu.sync_copy(acc_v, part_hbm.at[wid])


_sc_kernel = pl.kernel(
    _sc_body,
    out_type=jax.ShapeDtypeStruct((NW, 16), jnp.float32),
    mesh=plsc.VectorSubcoreMesh(core_axis_name="c", subcore_axis_name="s"),
    compiler_params=pltpu.CompilerParams(needs_layout_passes=False),
    scratch_types=[
        pltpu.VMEM((K,), jnp.int32),
        pltpu.VMEM((K,), jnp.int32),
        pltpu.VMEM((K,), jnp.int32),
        pltpu.VMEM((K,), jnp.float32),
        pltpu.VMEM((K,), jnp.float32),
        pltpu.VMEM((K,), jnp.float32),
        pltpu.VMEM((L * TB,), jnp.int32),
        pltpu.VMEM((L * TB,), jnp.float32),
        pltpu.VMEM((16,), jnp.float32),
        pltpu.SemaphoreType.DMA,
    ],
)


def _finish_body(part_ref, wrow_ref, cnt_ref, w_ref, out_ref):
    c0 = jnp.sum(w_ref[...] * (jnp.float32(G) + cnt_ref[...]))
    s = jnp.sum(part_ref[...] * wrow_ref[...])
    out_ref[...] = jnp.reshape(c0 - s, (1, 1))


_finish = pl.pallas_call(
    _finish_body,
    out_shape=jax.ShapeDtypeStruct((1, 1), jnp.float32),
)


def kernel(posterior_prob, latent_var_inds, latent_neg_mask, observed_neg_mask,
           observed_rule_cnts, rule_weights):
    tbl2 = jnp.concatenate([1.0 - posterior_prob, posterior_prob,
                            jnp.zeros((8,), jnp.float32)])
    sidx = latent_var_inds + latent_neg_mask * N_ATOMS
    obs_nz = jnp.sum(observed_neg_mask, axis=-1) != 0
    s0 = jnp.where(obs_nz, ZIDX, sidx[:, :, 0]).reshape(-1)
    s1 = sidx[:, :, 1].reshape(-1)
    s2 = sidx[:, :, 2].reshape(-1)

    partials = _sc_kernel(tbl2, s0, s1, s2)

    wrow = jnp.repeat(rule_weights[0], NW // R).reshape(NW, 1)
    out = _finish(partials, wrow, observed_rule_cnts.reshape(1, R),
                  rule_weights)
    return out.reshape(1)


# speedup vs baseline: 1.0001x; 1.0001x over previous
import jax
import jax.numpy as jnp
from jax import lax
from jax.experimental import pallas as pl
from jax.experimental.pallas import tpu as pltpu
from jax.experimental.pallas import tpu_sc as plsc

N_ATOMS = 1000000
R = 8
G = 100000
L = 3
O = 4
ZIDX = 2 * N_ATOMS
T2 = ZIDX + 8

NC = 2
NS = 16
NW = NC * NS
T = (R * G) // NW
K = 12480
NCH = 2
NGRP = K // 16
U = 6
TAIL = T - NCH * K
TGRP = 3
TB = TGRP * 16


def _sc_body(tbl2_hbm, s0_hbm, s1_hbm, s2_hbm, part_hbm,
             s0_v, s1_v, s2_v, p0_v, p1_v, p2_v,
             st_v, pt_v, acc_v, sem):
    wid = lax.axis_index("s") * NC + lax.axis_index("c")
    g0 = wid * T

    iota = lax.iota(jnp.int32, 16)
    zero_v = jnp.zeros((16,), jnp.float32)
    zero_i = jnp.zeros((16,), jnp.int32)

    st_v[pl.ds(L * TB - 16, 16)] = zero_i

    def chunk_body(ci, acc):
        b = g0 + ci * K
        pltpu.sync_copy(s0_hbm.at[pl.ds(b, K)], s0_v)
        pltpu.sync_copy(s1_hbm.at[pl.ds(b, K)], s1_v)
        pltpu.sync_copy(s2_hbm.at[pl.ds(b, K)], s2_v)
        pltpu.async_copy(tbl2_hbm.at[s0_v], p0_v, sem).wait()
        pltpu.async_copy(tbl2_hbm.at[s1_v], p1_v, sem).wait()
        pltpu.async_copy(tbl2_hbm.at[s2_v], p2_v, sem).wait()

        def grp(i, a):
            for u in range(U):
                ds_ = pl.ds((i * U + u) * 16, 16)
                a = a + p0_v[ds_] * p1_v[ds_] * p2_v[ds_]
            return a

        return lax.fori_loop(0, NGRP // U, grp, acc)

    acc = lax.fori_loop(0, NCH, chunk_body, zero_v)

    b = g0 + NCH * K
    pltpu.sync_copy(s0_hbm.at[pl.ds(b, TAIL)], st_v.at[pl.ds(0, TAIL)])
    pltpu.sync_copy(s1_hbm.at[pl.ds(b, TAIL)], st_v.at[pl.ds(TB, TAIL)])
    pltpu.sync_copy(s2_hbm.at[pl.ds(b, TAIL)], st_v.at[pl.ds(2 * TB, TAIL)])
    pltpu.async_copy(tbl2_hbm.at[st_v], pt_v, sem).wait()
    for j in range(TGRP):
        ds_ = pl.ds(j * 16, 16)
        prod = (pt_v[ds_] * pt_v[pl.ds(TB + j * 16, 16)]
                * pt_v[pl.ds(2 * TB + j * 16, 16)])
        valid = (j * 16 + iota) < TAIL
        acc = acc + jnp.where(valid, prod, zero_v)

    acc_v[...] = acc
    pltpu.sync_copy(acc_v, part_hbm.at[wid])


_sc_kernel = pl.kernel(
    _sc_body,
    out_type=jax.ShapeDtypeStruct((NW, 16), jnp.float32),
    mesh=plsc.VectorSubcoreMesh(core_axis_name="c", subcore_axis_name="s"),
    compiler_params=pltpu.CompilerParams(needs_layout_passes=False),
    scratch_types=[
        pltpu.VMEM((K,), jnp.int32),
        pltpu.VMEM((K,), jnp.int32),
        pltpu.VMEM((K,), jnp.int32),
        pltpu.VMEM((K,), jnp.float32),
        pltpu.VMEM((K,), jnp.float32),
        pltpu.VMEM((K,), jnp.float32),
        pltpu.VMEM((L * TB,), jnp.int32),
        pltpu.VMEM((L * TB,), jnp.float32),
        pltpu.VMEM((16,), jnp.float32),
        pltpu.SemaphoreType.DMA,
    ],
)


def _finish_body(part_ref, wrow_ref, cnt_ref, w_ref, out_ref):
    c0 = jnp.sum(w_ref[...] * (jnp.float32(G) + cnt_ref[...]))
    s = jnp.sum(part_ref[...] * wrow_ref[...])
    out_ref[...] = jnp.reshape(c0 - s, (1, 1))


_finish = pl.pallas_call(
    _finish_body,
    out_shape=jax.ShapeDtypeStruct((1, 1), jnp.float32),
)


def kernel(posterior_prob, latent_var_inds, latent_neg_mask, observed_neg_mask,
           observed_rule_cnts, rule_weights):
    tbl2 = jnp.concatenate([1.0 - posterior_prob, posterior_prob,
                            jnp.zeros((8,), jnp.float32)])
    sidx = latent_var_inds + latent_neg_mask * N_ATOMS
    obs_nz = jnp.sum(observed_neg_mask, axis=-1) != 0
    s0 = jnp.where(obs_nz, ZIDX, sidx[:, :, 0]).reshape(-1)
    s1 = sidx[:, :, 1].reshape(-1)
    s2 = sidx[:, :, 2].reshape(-1)

    partials = _sc_kernel(tbl2, s0, s1, s2)

    wrow = jnp.repeat(rule_weights[0], NW // R).reshape(NW, 1)
    out = _finish(partials, wrow, observed_rule_cnts.reshape(1, R),
                  rule_weights)
    return out.reshape(1)


# speedup vs baseline: 12.7699x; 12.7685x over previous
import jax
import jax.numpy as jnp
from jax import lax
from jax.experimental import pallas as pl
from jax.experimental.pallas import tpu as pltpu
from jax.experimental.pallas import tpu_sc as plsc

N_ATOMS = 1000000
R = 8
G = 100000
L = 3
O = 4
ZBASE = 2 * N_ATOMS

NC = 2
NS = 16
NW = NC * NS
T = (R * G) // NW
K = 12480
NCH = 2
NGRP = K // 16
U = 6
TAIL = T - NCH * K
TGRP = 3
TB = TGRP * 16


def _sc_body(tbl2_hbm, s0_hbm, s1_hbm, s2_hbm, part_hbm,
             s0_v, s1_v, s2_v, p0_v, p1_v, p2_v,
             st_v, pt_v, acc_v, sem):
    wid = lax.axis_index("s") * NC + lax.axis_index("c")
    g0 = wid * T

    iota = lax.iota(jnp.int32, 16)
    zero_v = jnp.zeros((16,), jnp.float32)
    zero_i = jnp.zeros((16,), jnp.int32)

    st_v[pl.ds(L * TB - 16, 16)] = zero_i

    def chunk_body(ci, acc):
        b = g0 + ci * K
        pltpu.sync_copy(s0_hbm.at[pl.ds(b, K)], s0_v)
        pltpu.sync_copy(s1_hbm.at[pl.ds(b, K)], s1_v)
        pltpu.sync_copy(s2_hbm.at[pl.ds(b, K)], s2_v)
        pltpu.async_copy(tbl2_hbm.at[s0_v], p0_v, sem).wait()
        pltpu.async_copy(tbl2_hbm.at[s1_v], p1_v, sem).wait()
        pltpu.async_copy(tbl2_hbm.at[s2_v], p2_v, sem).wait()

        def grp(i, a):
            for u in range(U):
                ds_ = pl.ds((i * U + u) * 16, 16)
                a = a + p0_v[ds_] * p1_v[ds_] * p2_v[ds_]
            return a

        return lax.fori_loop(0, NGRP // U, grp, acc)

    acc = lax.fori_loop(0, NCH, chunk_body, zero_v)

    b = g0 + NCH * K
    pltpu.sync_copy(s0_hbm.at[pl.ds(b, TAIL)], st_v.at[pl.ds(0, TAIL)])
    pltpu.sync_copy(s1_hbm.at[pl.ds(b, TAIL)], st_v.at[pl.ds(TB, TAIL)])
    pltpu.sync_copy(s2_hbm.at[pl.ds(b, TAIL)], st_v.at[pl.ds(2 * TB, TAIL)])
    pltpu.async_copy(tbl2_hbm.at[st_v], pt_v, sem).wait()
    for j in range(TGRP):
        ds_ = pl.ds(j * 16, 16)
        prod = (pt_v[ds_] * pt_v[pl.ds(TB + j * 16, 16)]
                * pt_v[pl.ds(2 * TB + j * 16, 16)])
        valid = (j * 16 + iota) < TAIL
        acc = acc + jnp.where(valid, prod, zero_v)

    acc_v[...] = acc
    pltpu.sync_copy(acc_v, part_hbm.at[wid])


_sc_kernel = pl.kernel(
    _sc_body,
    out_type=jax.ShapeDtypeStruct((NW, 16), jnp.float32),
    mesh=plsc.VectorSubcoreMesh(core_axis_name="c", subcore_axis_name="s"),
    compiler_params=pltpu.CompilerParams(needs_layout_passes=False),
    scratch_types=[
        pltpu.VMEM((K,), jnp.int32),
        pltpu.VMEM((K,), jnp.int32),
        pltpu.VMEM((K,), jnp.int32),
        pltpu.VMEM((K,), jnp.float32),
        pltpu.VMEM((K,), jnp.float32),
        pltpu.VMEM((K,), jnp.float32),
        pltpu.VMEM((L * TB,), jnp.int32),
        pltpu.VMEM((L * TB,), jnp.float32),
        pltpu.VMEM((16,), jnp.float32),
        pltpu.SemaphoreType.DMA,
    ],
)


def _finish_body(part_ref, wrow_ref, cnt_ref, w_ref, out_ref):
    c0 = jnp.sum(w_ref[...] * (jnp.float32(G) + cnt_ref[...]))
    s = jnp.sum(part_ref[...] * wrow_ref[...])
    out_ref[...] = jnp.reshape(c0 - s, (1, 1))


_finish = pl.pallas_call(
    _finish_body,
    out_shape=jax.ShapeDtypeStruct((1, 1), jnp.float32),
)


def kernel(posterior_prob, latent_var_inds, latent_neg_mask, observed_neg_mask,
           observed_rule_cnts, rule_weights):
    tbl2 = jnp.concatenate([1.0 - posterior_prob, posterior_prob,
                            jnp.zeros((N_ATOMS,), jnp.float32)])
    sidx = latent_var_inds + latent_neg_mask * N_ATOMS
    obs_nz = jnp.sum(observed_neg_mask, axis=-1) != 0
    s0 = jnp.where(obs_nz, ZBASE + latent_var_inds[:, :, 0],
                   sidx[:, :, 0]).reshape(-1)
    s1 = sidx[:, :, 1].reshape(-1)
    s2 = sidx[:, :, 2].reshape(-1)

    partials = _sc_kernel(tbl2, s0, s1, s2)

    wrow = jnp.repeat(rule_weights[0], NW // R).reshape(NW, 1)
    out = _finish(partials, wrow, observed_rule_cnts.reshape(1, R),
                  rule_weights)
    return out.reshape(1)


# speedup vs baseline: 19.2384x; 1.5065x over previous
import jax
import jax.numpy as jnp
from jax import lax
from jax.experimental import pallas as pl
from jax.experimental.pallas import tpu as pltpu
from jax.experimental.pallas import tpu_sc as plsc

N_ATOMS = 1000000
R = 8
G = 100000
L = 3
O = 4

NC = 2
NS = 16
NW = NC * NS
T = (R * G) // NW
K = 6240
NCH = 4
NGRP = K // 16
U = 6
TAIL = T - NCH * K
TGRP = 3
TB = TGRP * 16
FLAG = 1 << 21
IDXMASK = FLAG - 1


def _sc_body(tbl_hbm, pk0_hbm, pk1_hbm, pk2_hbm, part_hbm,
             pk_a0, pk_a1, pk_a2, pk_b0, pk_b1, pk_b2,
             idx_a, idx_b, p_a, p_b,
             pkt0_v, pkt1_v, pkt2_v, idxt_v, pt_v, acc_v,
             sem_a, sem_b, semg_a, semg_b):
    wid = lax.axis_index("s") * NC + lax.axis_index("c")
    g0 = wid * T

    iota = lax.iota(jnp.int32, 16)
    zero_v = jnp.zeros((16,), jnp.float32)
    zero_i = jnp.zeros((16,), jnp.int32)
    one_f = jnp.float32(1.0)

    pk_bufs = ((pk_a0, pk_a1, pk_a2), (pk_b0, pk_b1, pk_b2))
    idx_bufs = (idx_a, idx_b)
    p_bufs = (p_a, p_b)
    sems = (sem_a, sem_b)
    semgs = (semg_a, semg_b)

    pkt_refs = (pkt0_v, pkt1_v, pkt2_v)
    for l in range(L):
        pkt_refs[l][pl.ds(TB - 16, 16)] = zero_i

    def dma_start(ci, s):
        b = g0 + ci * K
        pk_v, sem = pk_bufs[s], sems[s]
        return [
            pltpu.async_copy(pk0_hbm.at[pl.ds(b, K)], pk_v[0], sem),
            pltpu.async_copy(pk1_hbm.at[pl.ds(b, K)], pk_v[1], sem),
            pltpu.async_copy(pk2_hbm.at[pl.ds(b, K)], pk_v[2], sem)]

    def unpack(s):
        pk_v, idx_v = pk_bufs[s], idx_bufs[s]

        def body(i, c):
            for u in range(U):
                j = i * U + u
                ds_ = pl.ds(j * 16, 16)
                v0 = pk_v[0][ds_]
                v1 = pk_v[1][ds_]
                v2 = pk_v[2][ds_]
                idx_v[pl.ds(j * 16, 16)] = (
                    lax.shift_right_logical(v0 & IDXMASK, 1))
                idx_v[pl.ds(K + j * 16, 16)] = lax.shift_right_logical(v1, 1)
                idx_v[pl.ds(2 * K + j * 16, 16)] = (
                    lax.shift_right_logical(v2, 1))
            return c
        lax.fori_loop(0, NGRP // U, body, 0)

    def accum(s, acc):
        pk_v, p_v = pk_bufs[s], p_bufs[s]

        def body(i, a):
            for u in range(U):
                j = i * U + u
                ds_ = pl.ds(j * 16, 16)
                v0 = pk_v[0][ds_]
                v1 = pk_v[1][ds_]
                v2 = pk_v[2][ds_]
                p0 = p_v[ds_]
                p1 = p_v[pl.ds(K + j * 16, 16)]
                p2 = p_v[pl.ds(2 * K + j * 16, 16)]
                sel0 = jnp.where((v0 & 1) == 1, p0, one_f - p0)
                sel1 = jnp.where((v1 & 1) == 1, p1, one_f - p1)
                sel2 = jnp.where((v2 & 1) == 1, p2, one_f - p2)
                prod = sel0 * sel1 * sel2
                a = a + jnp.where(v0 < FLAG, prod, zero_v)
            return a
        return lax.fori_loop(0, NGRP // U, body, acc)

    def gather_start(s):
        return pltpu.async_copy(tbl_hbm.at[idx_bufs[s]], p_bufs[s], semgs[s])

    acc = zero_v
    h = dma_start(0, 0)
    for x in h:
        x.wait()
    unpack(0)
    gh0 = gather_start(0)
    h = dma_start(1, 1)
    for x in h:
        x.wait()
    unpack(1)
    gh1 = gather_start(1)
    gh0.wait()
    acc = accum(0, acc)
    h = dma_start(2, 0)
    for x in h:
        x.wait()
    unpack(0)
    gh2 = gather_start(0)
    gh1.wait()
    acc = accum(1, acc)
    h = dma_start(3, 1)
    for x in h:
        x.wait()
    unpack(1)
    gh3 = gather_start(1)
    gh2.wait()
    acc = accum(0, acc)
    gh3.wait()
    acc = accum(1, acc)

    b = g0 + NCH * K
    pltpu.sync_copy(pk0_hbm.at[pl.ds(b, TAIL)], pkt0_v.at[pl.ds(0, TAIL)])
    pltpu.sync_copy(pk1_hbm.at[pl.ds(b, TAIL)], pkt1_v.at[pl.ds(0, TAIL)])
    pltpu.sync_copy(pk2_hbm.at[pl.ds(b, TAIL)], pkt2_v.at[pl.ds(0, TAIL)])
    for j in range(TGRP):
        ds_ = pl.ds(j * 16, 16)
        for l in range(L):
            v = pkt_refs[l][ds_]
            idxt_v[pl.ds(l * TB + j * 16, 16)] = (
                lax.shift_right_logical(v & IDXMASK, 1))
    pltpu.async_copy(tbl_hbm.at[idxt_v], pt_v, sem_a).wait()
    for j in range(TGRP):
        ds_ = pl.ds(j * 16, 16)
        prod = jnp.ones((16,), jnp.float32)
        v0 = pkt0_v[ds_]
        for l in range(L):
            v = pkt_refs[l][ds_] if l else v0
            pv = pt_v[pl.ds(l * TB + j * 16, 16)]
            prod = prod * jnp.where((v & 1) == 1, pv, one_f - pv)
        valid = ((j * 16 + iota) < TAIL) & (v0 < FLAG)
        acc = acc + jnp.where(valid, prod, zero_v)

    acc_v[...] = acc
    pltpu.sync_copy(acc_v, part_hbm.at[wid])


_sc_kernel = pl.kernel(
    _sc_body,
    out_type=jax.ShapeDtypeStruct((NW, 16), jnp.float32),
    mesh=plsc.VectorSubcoreMesh(core_axis_name="c", subcore_axis_name="s"),
    compiler_params=pltpu.CompilerParams(needs_layout_passes=False),
    scratch_types=[
        pltpu.VMEM((K,), jnp.int32),
        pltpu.VMEM((K,), jnp.int32),
        pltpu.VMEM((K,), jnp.int32),
        pltpu.VMEM((K,), jnp.int32),
        pltpu.VMEM((K,), jnp.int32),
        pltpu.VMEM((K,), jnp.int32),
        pltpu.VMEM((L * K,), jnp.int32),
        pltpu.VMEM((L * K,), jnp.int32),
        pltpu.VMEM((L * K,), jnp.float32),
        pltpu.VMEM((L * K,), jnp.float32),
        pltpu.VMEM((TB,), jnp.int32),
        pltpu.VMEM((TB,), jnp.int32),
        pltpu.VMEM((TB,), jnp.int32),
        pltpu.VMEM((L * TB,), jnp.int32),
        pltpu.VMEM((L * TB,), jnp.float32),
        pltpu.VMEM((16,), jnp.float32),
        pltpu.SemaphoreType.DMA,
        pltpu.SemaphoreType.DMA,
        pltpu.SemaphoreType.DMA,
        pltpu.SemaphoreType.DMA,
    ],
)


def _finish_body(part_ref, wrow_ref, cnt_ref, w_ref, out_ref):
    c0 = jnp.sum(w_ref[...] * (jnp.float32(G) + cnt_ref[...]))
    s = jnp.sum(part_ref[...] * wrow_ref[...])
    out_ref[...] = jnp.reshape(c0 - s, (1, 1))


_finish = pl.pallas_call(
    _finish_body,
    out_shape=jax.ShapeDtypeStruct((1, 1), jnp.float32),
)


def kernel(posterior_prob, latent_var_inds, latent_neg_mask, observed_neg_mask,
           observed_rule_cnts, rule_weights):
    base = latent_var_inds * 2 + latent_neg_mask
    obs_nz = jnp.sum(observed_neg_mask, axis=-1) != 0
    pk0 = (base[:, :, 0] + jnp.where(obs_nz, FLAG, 0)).reshape(-1)
    pk1 = base[:, :, 1].reshape(-1)
    pk2 = base[:, :, 2].reshape(-1)

    partials = _sc_kernel(posterior_prob, pk0, pk1, pk2)

    wrow = jnp.repeat(rule_weights[0], NW // R).reshape(NW, 1)
    out = _finish(partials, wrow, observed_rule_cnts.reshape(1, R),
                  rule_weights)
    return out.reshape(1)
